# Initial kernel scaffold; baseline (speedup 1.0000x reference)
import jax
import jax.numpy as jnp
from jax.experimental import pallas as pl


def kernel(x, Wq, Wk, Wv, Wo):
    raise NotImplementedError("write your pallas kernel here")


# speedup vs baseline: 1.3638x; 1.3638x over previous
import functools

import jax
import jax.numpy as jnp
import numpy as np
from jax.experimental import pallas as pl
from jax.experimental.pallas import tpu as pltpu

_H = 16
_NEG = -1e30


def _mm_kernel(x_ref, w_ref, o_ref):
    o_ref[...] = jnp.dot(
        x_ref[...], w_ref[...], preferred_element_type=jnp.float32
    ).astype(o_ref.dtype)


def _matmul(x, w, out_dtype, bm, bn):
    m, k = x.shape
    _, n = w.shape
    return pl.pallas_call(
        _mm_kernel,
        grid=(m // bm, n // bn),
        in_specs=[
            pl.BlockSpec((bm, k), lambda i, j: (i, 0)),
            pl.BlockSpec((k, bn), lambda i, j: (0, j)),
        ],
        out_specs=pl.BlockSpec((bm, bn), lambda i, j: (i, j)),
        out_shape=jax.ShapeDtypeStruct((m, n), out_dtype),
        compiler_params=pltpu.CompilerParams(
            dimension_semantics=("parallel", "parallel"),
            vmem_limit_bytes=48 * 1024 * 1024,
        ),
    )(x, w)


def _attn_kernel(q_ref, k_ref, v_ref, o_ref, *, bq, bk, scale):
    qi = pl.program_id(1)
    dk = q_ref.shape[2]
    q = q_ref[0]
    row = jax.lax.broadcasted_iota(jnp.int32, (bq, bk), 0) + qi * bq

    def body(j, carry):
        o_acc, m, l = carry
        off = pl.multiple_of(j * bk, bk)
        k_blk = k_ref[0, pl.ds(off, bk), :]
        s = jax.lax.dot_general(
            q, k_blk, (((1,), (1,)), ((), ())),
            preferred_element_type=jnp.float32,
        ) * scale
        col = jax.lax.broadcasted_iota(jnp.int32, (bq, bk), 1) + j * bk
        s = jnp.where(col <= row, s, _NEG)
        m_new = jnp.maximum(m, jnp.max(s, axis=-1, keepdims=True))
        alpha = jnp.exp(m - m_new)
        p = jnp.exp(s - m_new)
        l_new = l * alpha + jnp.sum(p, axis=-1, keepdims=True)
        v_blk = v_ref[0, pl.ds(off, bk), :]
        pv = jax.lax.dot_general(
            p.astype(jnp.bfloat16), v_blk, (((1,), (0,)), ((), ())),
            preferred_element_type=jnp.float32,
        )
        return o_acc * alpha + pv, m_new, l_new

    o0 = jnp.zeros((bq, dk), jnp.float32)
    m0 = jnp.full((bq, 1), _NEG, jnp.float32)
    l0 = jnp.zeros((bq, 1), jnp.float32)
    o_acc, _, l = jax.lax.fori_loop(0, qi + 1, body, (o0, m0, l0))
    o_ref[0] = (o_acc / l).astype(o_ref.dtype)


def _attention(qkv, bq, bk):
    B, S, threeD = qkv.shape
    D = threeD // 3
    dk = D // _H
    scale = 1.0 / np.sqrt(dk)
    kern = functools.partial(_attn_kernel, bq=bq, bk=bk, scale=scale)
    return pl.pallas_call(
        kern,
        grid=(B * _H, S // bq),
        in_specs=[
            pl.BlockSpec((1, bq, dk), lambda bh, qi: (bh // _H, qi, bh % _H)),
            pl.BlockSpec((1, S, dk), lambda bh, qi: (bh // _H, 0, _H + bh % _H)),
            pl.BlockSpec((1, S, dk), lambda bh, qi: (bh // _H, 0, 2 * _H + bh % _H)),
        ],
        out_specs=pl.BlockSpec((1, bq, dk), lambda bh, qi: (bh // _H, qi, bh % _H)),
        out_shape=jax.ShapeDtypeStruct((B, S, D), jnp.bfloat16),
        compiler_params=pltpu.CompilerParams(
            dimension_semantics=("parallel", "arbitrary"),
            vmem_limit_bytes=48 * 1024 * 1024,
        ),
    )(qkv, qkv, qkv)


def kernel(x, Wq, Wk, Wv, Wo):
    B, S, D = x.shape
    x2d = x.reshape(B * S, D).astype(jnp.bfloat16)
    Wqkv = jnp.concatenate([Wq, Wk, Wv], axis=1).astype(jnp.bfloat16)
    qkv = _matmul(x2d, Wqkv, jnp.bfloat16, 1024, 1024).reshape(B, S, 3 * D)
    o = _attention(qkv, bq=512, bk=512)
    out = _matmul(o.reshape(B * S, D), Wo.astype(jnp.bfloat16), jnp.float32,
                  1024, 1024)
    return out.reshape(B, S, D)


# speedup vs baseline: 1.5419x; 1.1305x over previous
import functools

import jax
import jax.numpy as jnp
import numpy as np
from jax.experimental import pallas as pl
from jax.experimental.pallas import tpu as pltpu

_H = 16
_NEG = -1e30
_QSCALE = float(np.log2(np.e) / np.sqrt(128.0))


def _qkv_kernel(x_ref, w_ref, o_ref):
    j = pl.program_id(1)
    acc = jnp.dot(x_ref[...], w_ref[...], preferred_element_type=jnp.float32)
    c = jnp.where(3 * j < pl.num_programs(1), _QSCALE, 1.0).astype(jnp.float32)
    o_ref[...] = (acc * c).astype(o_ref.dtype)


def _qkv_matmul(x, w, bm, bn):
    m, k = x.shape
    _, n = w.shape
    return pl.pallas_call(
        _qkv_kernel,
        grid=(m // bm, n // bn),
        in_specs=[
            pl.BlockSpec((bm, k), lambda i, j: (i, 0)),
            pl.BlockSpec((k, bn), lambda i, j: (0, j)),
        ],
        out_specs=pl.BlockSpec((bm, bn), lambda i, j: (i, j)),
        out_shape=jax.ShapeDtypeStruct((m, n), jnp.bfloat16),
        compiler_params=pltpu.CompilerParams(
            dimension_semantics=("parallel", "parallel"),
            vmem_limit_bytes=48 * 1024 * 1024,
        ),
    )(x, w)


def _attn_o_kernel(q_ref, k_ref, v_ref, wo_ref, o_ref, *, bq, bk):
    h = pl.program_id(1)
    S = q_ref.shape[1]
    nq = S // bq
    wo = wo_ref[...].astype(jnp.bfloat16)
    for qi in range(nq):
        q = q_ref[0, qi * bq:(qi + 1) * bq, :]
        o_acc = m = l = None
        for j in range(qi + 1):
            k_blk = k_ref[0, j * bk:(j + 1) * bk, :]
            s = jax.lax.dot_general(
                q, k_blk, (((1,), (1,)), ((), ())),
                preferred_element_type=jnp.float32,
            )
            if j == qi:
                rows = jax.lax.broadcasted_iota(jnp.int32, (bq, bk), 0)
                cols = jax.lax.broadcasted_iota(jnp.int32, (bq, bk), 1)
                s = jnp.where(cols <= rows, s, _NEG)
            v_blk = v_ref[0, j * bk:(j + 1) * bk, :]
            if j == 0:
                m = jnp.max(s, axis=-1, keepdims=True)
                p = jnp.exp2(s - m)
                l = jnp.sum(p, axis=-1, keepdims=True)
                o_acc = jax.lax.dot_general(
                    p.astype(jnp.bfloat16), v_blk, (((1,), (0,)), ((), ())),
                    preferred_element_type=jnp.float32,
                )
            else:
                m_new = jnp.maximum(m, jnp.max(s, axis=-1, keepdims=True))
                alpha = jnp.exp2(m - m_new)
                p = jnp.exp2(s - m_new)
                l = l * alpha + jnp.sum(p, axis=-1, keepdims=True)
                pv = jax.lax.dot_general(
                    p.astype(jnp.bfloat16), v_blk, (((1,), (0,)), ((), ())),
                    preferred_element_type=jnp.float32,
                )
                o_acc = o_acc * alpha + pv
                m = m_new
        o_h = (o_acc / l).astype(jnp.bfloat16)
        contrib = jax.lax.dot_general(
            o_h, wo, (((1,), (0,)), ((), ())),
            preferred_element_type=jnp.float32,
        )
        sl = slice(qi * bq, (qi + 1) * bq)

        @pl.when(h == 0)
        def _():
            o_ref[0, sl, :] = contrib

        @pl.when(h != 0)
        def _():
            o_ref[0, sl, :] = o_ref[0, sl, :] + contrib


def _attention_proj(qkv, Wo, bq, bk):
    B, S, threeD = qkv.shape
    D = threeD // 3
    dk = D // _H
    kern = functools.partial(_attn_o_kernel, bq=bq, bk=bk)
    return pl.pallas_call(
        kern,
        grid=(B, _H),
        in_specs=[
            pl.BlockSpec((1, S, dk), lambda b, h: (b, 0, h)),
            pl.BlockSpec((1, S, dk), lambda b, h: (b, 0, _H + h)),
            pl.BlockSpec((1, S, dk), lambda b, h: (b, 0, 2 * _H + h)),
            pl.BlockSpec((dk, D), lambda b, h: (h, 0)),
        ],
        out_specs=pl.BlockSpec((1, S, D), lambda b, h: (b, 0, 0)),
        out_shape=jax.ShapeDtypeStruct((B, S, D), jnp.float32),
        compiler_params=pltpu.CompilerParams(
            dimension_semantics=("parallel", "arbitrary"),
            vmem_limit_bytes=52 * 1024 * 1024,
        ),
    )(qkv, qkv, qkv, Wo)


def kernel(x, Wq, Wk, Wv, Wo):
    B, S, D = x.shape
    x2d = x.reshape(B * S, D).astype(jnp.bfloat16)
    Wqkv = jnp.concatenate([Wq, Wk, Wv], axis=1).astype(jnp.bfloat16)
    qkv = _qkv_matmul(x2d, Wqkv, 1024, 1024).reshape(B, S, 3 * D)
    return _attention_proj(qkv, Wo, bq=512, bk=512)


# speedup vs baseline: 2.2937x; 1.4876x over previous
import functools

import jax
import jax.numpy as jnp
import numpy as np
from jax.experimental import pallas as pl
from jax.experimental.pallas import tpu as pltpu

_H = 16
_NEG = -1e30
_QSCALE = float(np.log2(np.e) / np.sqrt(128.0))


def _qkv_kernel(x_ref, wq_ref, wk_ref, wv_ref, q_ref, k_ref, v_ref):
    x = x_ref[...].astype(jnp.bfloat16)
    q_ref[...] = (jnp.dot(x, wq_ref[...].astype(jnp.bfloat16),
                          preferred_element_type=jnp.float32)
                  * _QSCALE).astype(jnp.bfloat16)
    k_ref[...] = jnp.dot(x, wk_ref[...].astype(jnp.bfloat16),
                         preferred_element_type=jnp.float32).astype(jnp.bfloat16)
    v_ref[...] = jnp.dot(x, wv_ref[...].astype(jnp.bfloat16),
                         preferred_element_type=jnp.float32).astype(jnp.bfloat16)


def _qkv_matmul(x, wq, wk, wv, bm, bn):
    m, d = x.shape
    grid = (m // bm, d // bn)
    wspec = pl.BlockSpec((d, bn), lambda i, j: (0, j))
    ospec = pl.BlockSpec((bm, bn), lambda i, j: (i, j))
    osds = jax.ShapeDtypeStruct((m, d), jnp.bfloat16)
    return pl.pallas_call(
        _qkv_kernel,
        grid=grid,
        in_specs=[pl.BlockSpec((bm, d), lambda i, j: (i, 0)),
                  wspec, wspec, wspec],
        out_specs=[ospec, ospec, ospec],
        out_shape=[osds, osds, osds],
        compiler_params=pltpu.CompilerParams(
            dimension_semantics=("parallel", "parallel"),
            vmem_limit_bytes=57 * 1024 * 1024,
        ),
    )(x, wq, wk, wv)


def _attn_kernel(q_ref, k_ref, v_ref, o_ref, *, bq, bk):
    S = q_ref.shape[1]
    nq = S // bq
    for qi in range(nq):
        q = q_ref[0, qi * bq:(qi + 1) * bq, :]
        o_acc = m = l = None
        for j in range(qi + 1):
            k_blk = k_ref[0, j * bk:(j + 1) * bk, :]
            s = jax.lax.dot_general(
                q, k_blk, (((1,), (1,)), ((), ())),
                preferred_element_type=jnp.float32,
            )
            if j == qi:
                rows = jax.lax.broadcasted_iota(jnp.int32, (bq, bk), 0)
                cols = jax.lax.broadcasted_iota(jnp.int32, (bq, bk), 1)
                s = jnp.where(cols <= rows, s, _NEG)
            v_blk = v_ref[0, j * bk:(j + 1) * bk, :]
            if j == 0:
                m = jnp.max(s, axis=-1, keepdims=True)
                p = jnp.exp2(s - m)
                l = jnp.sum(p, axis=-1, keepdims=True)
                o_acc = jax.lax.dot_general(
                    p.astype(jnp.bfloat16), v_blk, (((1,), (0,)), ((), ())),
                    preferred_element_type=jnp.float32,
                )
            else:
                m_new = jnp.maximum(m, jnp.max(s, axis=-1, keepdims=True))
                alpha = jnp.exp2(m - m_new)
                p = jnp.exp2(s - m_new)
                l = l * alpha + jnp.sum(p, axis=-1, keepdims=True)
                pv = jax.lax.dot_general(
                    p.astype(jnp.bfloat16), v_blk, (((1,), (0,)), ((), ())),
                    preferred_element_type=jnp.float32,
                )
                o_acc = o_acc * alpha + pv
                m = m_new
        o_ref[0, qi * bq:(qi + 1) * bq, :] = (o_acc / l).astype(jnp.bfloat16)


def _attention(q, k, v, bq, bk):
    B, S, D = q.shape
    dk = D // _H
    kern = functools.partial(_attn_kernel, bq=bq, bk=bk)
    hspec = pl.BlockSpec((1, S, dk), lambda b, h: (b, 0, h))
    return pl.pallas_call(
        kern,
        grid=(B, _H),
        in_specs=[hspec, hspec, hspec],
        out_specs=hspec,
        out_shape=jax.ShapeDtypeStruct((B, S, D), jnp.bfloat16),
        compiler_params=pltpu.CompilerParams(
            dimension_semantics=("parallel", "arbitrary"),
            vmem_limit_bytes=50 * 1024 * 1024,
        ),
    )(q, k, v)


def _out_kernel(o_ref, w_ref, out_ref):
    out_ref[...] = jnp.dot(o_ref[...], w_ref[...].astype(jnp.bfloat16),
                           preferred_element_type=jnp.float32)


def _out_matmul(o, w, bm, bn):
    m, d = o.shape
    return pl.pallas_call(
        _out_kernel,
        grid=(m // bm, d // bn),
        in_specs=[pl.BlockSpec((bm, d), lambda i, j: (i, 0)),
                  pl.BlockSpec((d, bn), lambda i, j: (0, j))],
        out_specs=pl.BlockSpec((bm, bn), lambda i, j: (i, j)),
        out_shape=jax.ShapeDtypeStruct((m, d), jnp.float32),
        compiler_params=pltpu.CompilerParams(
            dimension_semantics=("parallel", "parallel"),
            vmem_limit_bytes=50 * 1024 * 1024,
        ),
    )(o, w)


def kernel(x, Wq, Wk, Wv, Wo):
    B, S, D = x.shape
    x2d = x.reshape(B * S, D).astype(jnp.bfloat16)
    q, k, v = _qkv_matmul(x2d, Wq, Wk, Wv, bm=2048, bn=256)
    q = q.reshape(B, S, D)
    k = k.reshape(B, S, D)
    v = v.reshape(B, S, D)
    o = _attention(q, k, v, bq=512, bk=512)
    out = _out_matmul(o.reshape(B * S, D), Wo, bm=2048, bn=512)
    return out.reshape(B, S, D)


# speedup vs baseline: 2.8698x; 1.2512x over previous
import functools

import jax
import jax.numpy as jnp
import numpy as np
from jax.experimental import pallas as pl
from jax.experimental.pallas import tpu as pltpu

_H = 16
_NEG = -1e30
_QSCALE = float(np.log2(np.e) / np.sqrt(128.0))


def _qkv_kernel(x_ref, wq_ref, wk_ref, wv_ref, q_ref, k_ref, v_ref):
    x = x_ref[...].astype(jnp.bfloat16)
    q_ref[...] = (jnp.dot(x, wq_ref[...].astype(jnp.bfloat16),
                          preferred_element_type=jnp.float32)
                  * _QSCALE).astype(jnp.bfloat16)
    k_ref[...] = jnp.dot(x, wk_ref[...].astype(jnp.bfloat16),
                         preferred_element_type=jnp.float32).astype(jnp.bfloat16)
    v_ref[...] = jnp.dot(x, wv_ref[...].astype(jnp.bfloat16),
                         preferred_element_type=jnp.float32).astype(jnp.bfloat16)


def _qkv_matmul(x, wq, wk, wv, bm, bn):
    m, d = x.shape
    grid = (m // bm, d // bn)
    wspec = pl.BlockSpec((d, bn), lambda i, j: (0, j))
    ospec = pl.BlockSpec((bm, bn), lambda i, j: (i, j))
    osds = jax.ShapeDtypeStruct((m, d), jnp.bfloat16)
    return pl.pallas_call(
        _qkv_kernel,
        grid=grid,
        in_specs=[pl.BlockSpec((bm, d), lambda i, j: (i, 0)),
                  wspec, wspec, wspec],
        out_specs=[ospec, ospec, ospec],
        out_shape=[osds, osds, osds],
        compiler_params=pltpu.CompilerParams(
            dimension_semantics=("parallel", "parallel"),
            vmem_limit_bytes=57 * 1024 * 1024,
        ),
    )(x, wq, wk, wv)


def _attn_kernel(q_ref, k_ref, v_ref, o_ref, *, bq, bk):
    S = q_ref.shape[1]
    nq = S // bq
    for qi in range(nq):
        q = q_ref[0, qi * bq:(qi + 1) * bq, :]
        o_acc = m = l = None
        for j in range(qi + 1):
            k_blk = k_ref[0, j * bk:(j + 1) * bk, :]
            s = jax.lax.dot_general(
                q, k_blk, (((1,), (1,)), ((), ())),
                preferred_element_type=jnp.float32,
            )
            if j == qi:
                rows = jax.lax.broadcasted_iota(jnp.int32, (bq, bk), 0)
                cols = jax.lax.broadcasted_iota(jnp.int32, (bq, bk), 1)
                s = jnp.where(cols <= rows, s, _NEG)
            v_blk = v_ref[0, j * bk:(j + 1) * bk, :]
            if j == 0:
                m = jnp.max(s, axis=-1, keepdims=True)
                p = jnp.exp2(s - m)
                l = jnp.sum(p, axis=-1, keepdims=True)
                o_acc = jax.lax.dot_general(
                    p.astype(jnp.bfloat16), v_blk, (((1,), (0,)), ((), ())),
                    preferred_element_type=jnp.float32,
                )
            else:
                m_new = jnp.maximum(m, jnp.max(s, axis=-1, keepdims=True))
                alpha = jnp.exp2(m - m_new)
                p = jnp.exp2(s - m_new)
                l = l * alpha + jnp.sum(p, axis=-1, keepdims=True)
                pv = jax.lax.dot_general(
                    p.astype(jnp.bfloat16), v_blk, (((1,), (0,)), ((), ())),
                    preferred_element_type=jnp.float32,
                )
                o_acc = o_acc * alpha + pv
                m = m_new
        o_ref[0, qi * bq:(qi + 1) * bq, :] = (o_acc / l).astype(jnp.bfloat16)


def _attention(q, k, v, bq, bk):
    B, S, D = q.shape
    dk = D // _H
    kern = functools.partial(_attn_kernel, bq=bq, bk=bk)
    hspec = pl.BlockSpec((1, S, dk), lambda b, h: (b, 0, h))
    return pl.pallas_call(
        kern,
        grid=(B, _H),
        in_specs=[hspec, hspec, hspec],
        out_specs=hspec,
        out_shape=jax.ShapeDtypeStruct((B, S, D), jnp.bfloat16),
        compiler_params=pltpu.CompilerParams(
            dimension_semantics=("parallel", "arbitrary"),
            vmem_limit_bytes=50 * 1024 * 1024,
        ),
    )(q, k, v)


def _out_kernel(o_ref, w_ref, out_ref):
    out_ref[...] = jnp.dot(o_ref[...], w_ref[...].astype(jnp.bfloat16),
                           preferred_element_type=jnp.float32)


def _out_matmul(o, w, bm, bn):
    m, d = o.shape
    return pl.pallas_call(
        _out_kernel,
        grid=(m // bm, d // bn),
        in_specs=[pl.BlockSpec((bm, d), lambda i, j: (i, 0)),
                  pl.BlockSpec((d, bn), lambda i, j: (0, j))],
        out_specs=pl.BlockSpec((bm, bn), lambda i, j: (i, j)),
        out_shape=jax.ShapeDtypeStruct((m, d), jnp.float32),
        compiler_params=pltpu.CompilerParams(
            dimension_semantics=("parallel", "parallel"),
            vmem_limit_bytes=50 * 1024 * 1024,
        ),
    )(o, w)


def kernel(x, Wq, Wk, Wv, Wo):
    B, S, D = x.shape
    x2d = x.reshape(B * S, D).astype(jnp.bfloat16)
    q, k, v = _qkv_matmul(x2d, Wq, Wk, Wv, bm=2048, bn=256)
    q = q.reshape(B, S, D)
    k = k.reshape(B, S, D)
    v = v.reshape(B, S, D)
    o = _attention(q, k, v, bq=512, bk=256)
    out = _out_matmul(o.reshape(B * S, D), Wo, bm=2048, bn=512)
    return out.reshape(B, S, D)
